# Initial kernel scaffold; baseline (speedup 1.0000x reference)
import jax
import jax.numpy as jnp
from jax.experimental import pallas as pl


def kernel(feat_index, feat_value, emb_W, w1, fm_bias, W0, b0, W1, b1, W2, b2, Wo, bo):
    raise NotImplementedError("write your pallas kernel here")


# speedup vs baseline: 1.3088x; 1.3088x over previous
import functools

import numpy as np
import jax
import jax.numpy as jnp
from jax import lax
from jax.experimental import pallas as pl
from jax.experimental.pallas import tpu as pltpu
from jax.experimental.pallas import tpu_sc as plsc

B = 16384
F = 26
E = 16
D_IN = F * E
BF = B * F
NW = 32
PER_W = BF // NW
K = 1664
NCHUNK = PER_W // K

@functools.lru_cache(maxsize=None)
def _sc_gather_fn():
    mesh = plsc.VectorSubcoreMesh(core_axis_name="c", subcore_axis_name="s")

    @functools.partial(
        pl.kernel,
        mesh=mesh,
        compiler_params=pltpu.CompilerParams(use_tc_tiling_on_sc=False),
        out_type=(
            jax.ShapeDtypeStruct((BF, E), jnp.float32),
            jax.ShapeDtypeStruct((BF,), jnp.float32),
        ),
        scratch_types=[
            pltpu.VMEM((K,), jnp.int32),
            pltpu.VMEM((K,), jnp.int32),
            pltpu.VMEM((K, E), jnp.float32),
            pltpu.VMEM((K, E), jnp.float32),
            pltpu.VMEM((K,), jnp.float32),
            pltpu.VMEM((K,), jnp.float32),
            pltpu.SemaphoreType.DMA,
            pltpu.SemaphoreType.DMA,
        ],
    )
    def _sc_gather(idx_hbm, emb_hbm, w1_hbm, emb_out, w1_out,
                   idx_a, idx_b, rows_a, rows_b, w1a, w1b, sem_r, sem_w):
        wid = lax.axis_index("s") * 2 + lax.axis_index("c")
        base = wid * PER_W
        idx_bufs = (idx_a, idx_b)
        row_bufs = (rows_a, rows_b)
        w1_bufs = (w1a, w1b)

        def load_idx(c):
            pltpu.sync_copy(idx_hbm.at[pl.ds(base + c * K, K)], idx_bufs[c % 2])

        def start_gather(c):
            h1 = pltpu.async_copy(emb_hbm.at[idx_bufs[c % 2]], row_bufs[c % 2], sem_r)
            h2 = pltpu.async_copy(w1_hbm.at[idx_bufs[c % 2]], w1_bufs[c % 2], sem_w)
            return (h1, h2)

        load_idx(0)
        pend = start_gather(0)
        for c in range(NCHUNK):
            if c + 1 < NCHUNK:
                load_idx(c + 1)
            for h in pend:
                h.wait()
            if c + 1 < NCHUNK:
                pend = start_gather(c + 1)
            pltpu.sync_copy(row_bufs[c % 2], emb_out.at[pl.ds(base + c * K, K)])
            pltpu.sync_copy(w1_bufs[c % 2], w1_out.at[pl.ds(base + c * K, K)])

    return _sc_gather


_R_np = np.kron(np.eye(F, dtype=np.float32), np.ones((1, E), dtype=np.float32))
_G_np = np.tile(np.eye(E, dtype=np.float32), (F, 1))

BM = 1024
GRID = B // BM


def _tc_body(x_ref, v_ref, w1g_ref, r_ref, g_ref, w0_ref, b0_ref, w1_ref, b1_ref,
             w2_ref, b2_ref, wo_ref, scal_ref, o_ref):
    x = x_ref[...]
    v = v_ref[...]
    w1g = w1g_ref[...]
    fm_bias = scal_ref[0, 0]
    wo0 = scal_ref[0, 1]
    bo0 = scal_ref[0, 2]
    y1 = jnp.sum(w1g * v, axis=1, keepdims=True)
    vexp = jnp.dot(v, r_ref[...], preferred_element_type=jnp.float32)
    ev = x * vexp
    s = jnp.dot(ev, g_ref[...], preferred_element_type=jnp.float32)
    sq = jnp.dot(ev * ev, g_ref[...], preferred_element_type=jnp.float32)
    y2 = 0.5 * (jnp.sum(s * s, axis=1, keepdims=True)
                - jnp.sum(sq, axis=1, keepdims=True))
    yfm = y1 + y2 + fm_bias
    h = jnp.maximum(jnp.dot(x, w0_ref[...], preferred_element_type=jnp.float32)
                    + b0_ref[...], 0.0)
    h = jnp.maximum(jnp.dot(h, w1_ref[...], preferred_element_type=jnp.float32)
                    + b1_ref[...], 0.0)
    h = jnp.maximum(jnp.dot(h, w2_ref[...], preferred_element_type=jnp.float32)
                    + b2_ref[...], 0.0)
    z = yfm * wo0 + jnp.dot(h, wo_ref[...], preferred_element_type=jnp.float32) + bo0
    o_ref[...] = jax.nn.sigmoid(z)


def kernel(feat_index, feat_value, emb_W, w1, fm_bias, W0, b0, W1, b1, W2, b2, Wo, bo):
    idx_flat = feat_index.reshape(-1).astype(jnp.int32)
    emb_rows, w1g = _sc_gather_fn()(idx_flat, emb_W, w1.reshape(-1))
    x = emb_rows.reshape(B, D_IN)
    w1g2 = w1g.reshape(B, F)
    scal = jnp.stack([fm_bias.astype(jnp.float32), Wo[0, 0], bo[0]]).reshape(1, 3)
    out = pl.pallas_call(
        _tc_body,
        grid=(GRID,),
        in_specs=[
            pl.BlockSpec((BM, D_IN), lambda i: (i, 0)),
            pl.BlockSpec((BM, F), lambda i: (i, 0)),
            pl.BlockSpec((BM, F), lambda i: (i, 0)),
            pl.BlockSpec((F, D_IN), lambda i: (0, 0)),
            pl.BlockSpec((D_IN, E), lambda i: (0, 0)),
            pl.BlockSpec((D_IN, 32), lambda i: (0, 0)),
            pl.BlockSpec((1, 32), lambda i: (0, 0)),
            pl.BlockSpec((32, 32), lambda i: (0, 0)),
            pl.BlockSpec((1, 32), lambda i: (0, 0)),
            pl.BlockSpec((32, 32), lambda i: (0, 0)),
            pl.BlockSpec((1, 32), lambda i: (0, 0)),
            pl.BlockSpec((32, 1), lambda i: (0, 0)),
            pl.BlockSpec((1, 3), lambda i: (0, 0)),
        ],
        out_specs=pl.BlockSpec((BM, 1), lambda i: (i, 0)),
        out_shape=jax.ShapeDtypeStruct((B, 1), jnp.float32),
    )(x, feat_value, w1g2, jnp.asarray(_R_np), jnp.asarray(_G_np),
      W0, b0.reshape(1, 32), W1, b1.reshape(1, 32), W2, b2.reshape(1, 32),
      Wo[1:, :], scal)
    return out


# speedup vs baseline: 1.5085x; 1.1526x over previous
import functools

import numpy as np
import jax
import jax.numpy as jnp
from jax import lax
from jax.experimental import pallas as pl
from jax.experimental.pallas import tpu as pltpu
from jax.experimental.pallas import tpu_sc as plsc

B = 16384
F = 26
E = 16
FEAT_DIM = 1000000
D_IN = F * E
BF = B * F
NW = 32
PER_W = BF // NW
K = 1664
NCHUNK = PER_W // K

@functools.lru_cache(maxsize=None)
def _sc_gather_fn():
    mesh = plsc.VectorSubcoreMesh(core_axis_name="c", subcore_axis_name="s")

    @functools.partial(
        pl.kernel,
        mesh=mesh,
        compiler_params=pltpu.CompilerParams(use_tc_tiling_on_sc=False),
        out_type=(
            jax.ShapeDtypeStruct((BF, E), jnp.float32),
            jax.ShapeDtypeStruct((BF,), jnp.float32),
        ),
        scratch_types=[
            pltpu.VMEM((K,), jnp.int32),
            pltpu.VMEM((K,), jnp.int32),
            pltpu.VMEM((K, E), jnp.float32),
            pltpu.VMEM((K, E), jnp.float32),
            pltpu.VMEM((K,), jnp.float32),
            pltpu.VMEM((K,), jnp.float32),
            pltpu.SemaphoreType.DMA,
            pltpu.SemaphoreType.DMA,
        ],
    )
    def _sc_gather(idx_hbm, emb_hbm, w1_hbm, emb_out, w1_out,
                   idx_a, idx_b, rows_a, rows_b, w1a, w1b, sem_r, sem_w):
        wid = lax.axis_index("s") * 2 + lax.axis_index("c")
        base = wid * PER_W
        idx_bufs = (idx_a, idx_b)
        row_bufs = (rows_a, rows_b)
        w1_bufs = (w1a, w1b)

        def load_idx(c):
            pltpu.sync_copy(idx_hbm.at[pl.ds(base + c * K, K)], idx_bufs[c % 2])

        def start_gather(c):
            h1 = pltpu.async_copy(emb_hbm.at[idx_bufs[c % 2]], row_bufs[c % 2], sem_r)
            h2 = pltpu.async_copy(w1_hbm.at[idx_bufs[c % 2]], w1_bufs[c % 2], sem_w)
            return (h1, h2)

        load_idx(0)
        pend = start_gather(0)
        for c in range(NCHUNK):
            if c + 1 < NCHUNK:
                load_idx(c + 1)
            for h in pend:
                h.wait()
            if c + 1 < NCHUNK:
                pend = start_gather(c + 1)
            pltpu.sync_copy(row_bufs[c % 2], emb_out.at[pl.ds(base + c * K, K)])
            pltpu.sync_copy(w1_bufs[c % 2], w1_out.at[pl.ds(base + c * K, K)])

    return _sc_gather


CP = 8192
PREP_GRID = -(-FEAT_DIM // CP)


def _prep_body(xt_ref, o_ref):
    z3 = xt_ref[...].T.reshape(CP // 8, 8, E)
    o_ref[...] = jnp.concatenate([z3[:, j, :] for j in range(8)], axis=1)


_R_np = np.kron(np.eye(F, dtype=np.float32), np.ones((1, E), dtype=np.float32))
_G_np = np.tile(np.eye(E, dtype=np.float32), (F, 1))

BM = 1024
GRID = B // BM


def _tc_body(x_ref, v_ref, w1g_ref, r_ref, g_ref, w0_ref, b0_ref, w1_ref, b1_ref,
             w2_ref, b2_ref, wo_ref, scal_ref, o_ref):
    x = x_ref[...]
    v = v_ref[...]
    w1g = w1g_ref[...]
    fm_bias = scal_ref[0, 0]
    wo0 = scal_ref[0, 1]
    bo0 = scal_ref[0, 2]
    y1 = jnp.sum(w1g * v, axis=1, keepdims=True)
    vexp = jnp.dot(v, r_ref[...], preferred_element_type=jnp.float32)
    ev = x * vexp
    s = jnp.dot(ev, g_ref[...], preferred_element_type=jnp.float32)
    sq = jnp.dot(ev * ev, g_ref[...], preferred_element_type=jnp.float32)
    y2 = 0.5 * (jnp.sum(s * s, axis=1, keepdims=True)
                - jnp.sum(sq, axis=1, keepdims=True))
    yfm = y1 + y2 + fm_bias
    h = jnp.maximum(jnp.dot(x, w0_ref[...], preferred_element_type=jnp.float32)
                    + b0_ref[...], 0.0)
    h = jnp.maximum(jnp.dot(h, w1_ref[...], preferred_element_type=jnp.float32)
                    + b1_ref[...], 0.0)
    h = jnp.maximum(jnp.dot(h, w2_ref[...], preferred_element_type=jnp.float32)
                    + b2_ref[...], 0.0)
    z = yfm * wo0 + jnp.dot(h, wo_ref[...], preferred_element_type=jnp.float32) + bo0
    o_ref[...] = jax.nn.sigmoid(z)


def kernel(feat_index, feat_value, emb_W, w1, fm_bias, W0, b0, W1, b1, W2, b2, Wo, bo):
    idx_flat = feat_index.reshape(-1).astype(jnp.int32)
    emb128 = pl.pallas_call(
        _prep_body,
        grid=(PREP_GRID,),
        in_specs=[pl.BlockSpec((E, CP), lambda i: (0, i))],
        out_specs=pl.BlockSpec((CP // 8, 128), lambda i: (i, 0)),
        out_shape=jax.ShapeDtypeStruct((FEAT_DIM // 8, 128), jnp.float32),
    )(emb_W.T)
    emb_lin = emb128.reshape(FEAT_DIM, E)
    emb_rows, w1g = _sc_gather_fn()(idx_flat, emb_lin, w1.reshape(-1))
    x = emb_rows.reshape(B, D_IN)
    w1g2 = w1g.reshape(B, F)
    scal = jnp.stack([fm_bias.astype(jnp.float32), Wo[0, 0], bo[0]]).reshape(1, 3)
    out = pl.pallas_call(
        _tc_body,
        grid=(GRID,),
        in_specs=[
            pl.BlockSpec((BM, D_IN), lambda i: (i, 0)),
            pl.BlockSpec((BM, F), lambda i: (i, 0)),
            pl.BlockSpec((BM, F), lambda i: (i, 0)),
            pl.BlockSpec((F, D_IN), lambda i: (0, 0)),
            pl.BlockSpec((D_IN, E), lambda i: (0, 0)),
            pl.BlockSpec((D_IN, 32), lambda i: (0, 0)),
            pl.BlockSpec((1, 32), lambda i: (0, 0)),
            pl.BlockSpec((32, 32), lambda i: (0, 0)),
            pl.BlockSpec((1, 32), lambda i: (0, 0)),
            pl.BlockSpec((32, 32), lambda i: (0, 0)),
            pl.BlockSpec((1, 32), lambda i: (0, 0)),
            pl.BlockSpec((32, 1), lambda i: (0, 0)),
            pl.BlockSpec((1, 3), lambda i: (0, 0)),
        ],
        out_specs=pl.BlockSpec((BM, 1), lambda i: (i, 0)),
        out_shape=jax.ShapeDtypeStruct((B, 1), jnp.float32),
    )(x, feat_value, w1g2, jnp.asarray(_R_np), jnp.asarray(_G_np),
      W0, b0.reshape(1, 32), W1, b1.reshape(1, 32), W2, b2.reshape(1, 32),
      Wo[1:, :], scal)
    return out


# speedup vs baseline: 1.7367x; 1.1513x over previous
import functools

import numpy as np
import jax
import jax.numpy as jnp
from jax import lax
from jax.experimental import pallas as pl
from jax.experimental.pallas import tpu as pltpu
from jax.experimental.pallas import tpu_sc as plsc

B = 16384
F = 26
E = 16
FEAT_DIM = 1000000
D_IN = F * E
BF = B * F
NW = 32
PER_W = BF // NW
K = 1664
NCHUNK = PER_W // K

@functools.lru_cache(maxsize=None)
def _sc_gather_fn():
    mesh = plsc.VectorSubcoreMesh(core_axis_name="c", subcore_axis_name="s")

    @functools.partial(
        pl.kernel,
        mesh=mesh,
        compiler_params=pltpu.CompilerParams(use_tc_tiling_on_sc=False),
        out_type=(
            jax.ShapeDtypeStruct((BF, E), jnp.float32),
            jax.ShapeDtypeStruct((BF,), jnp.float32),
        ),
        scratch_types=[
            pltpu.VMEM((K,), jnp.int32),
            pltpu.VMEM((K,), jnp.int32),
            pltpu.VMEM((K, E), jnp.float32),
            pltpu.VMEM((K, E), jnp.float32),
            pltpu.VMEM((K,), jnp.float32),
            pltpu.VMEM((K,), jnp.float32),
            pltpu.SemaphoreType.DMA,
            pltpu.SemaphoreType.DMA,
        ],
    )
    def _sc_gather(idx_hbm, emb_hbm, w1_hbm, emb_out, w1_out,
                   idx_a, idx_b, rows_a, rows_b, w1a, w1b, sem_r, sem_w):
        wid = lax.axis_index("s") * 2 + lax.axis_index("c")
        base = wid * PER_W
        idx_bufs = (idx_a, idx_b)
        row_bufs = (rows_a, rows_b)
        w1_bufs = (w1a, w1b)

        def load_idx(c):
            pltpu.sync_copy(idx_hbm.at[pl.ds(base + c * K, K)], idx_bufs[c % 2])

        def start_gather(c):
            h1 = pltpu.async_copy(emb_hbm.at[idx_bufs[c % 2]], row_bufs[c % 2], sem_r)
            h2 = pltpu.async_copy(w1_hbm.at[idx_bufs[c % 2]], w1_bufs[c % 2], sem_w)
            return (h1, h2)

        load_idx(0)
        pend = start_gather(0)
        for c in range(NCHUNK):
            if c + 1 < NCHUNK:
                load_idx(c + 1)
            for h in pend:
                h.wait()
            if c + 1 < NCHUNK:
                pend = start_gather(c + 1)
            pltpu.sync_copy(row_bufs[c % 2], emb_out.at[pl.ds(base + c * K, K)])
            pltpu.sync_copy(w1_bufs[c % 2], w1_out.at[pl.ds(base + c * K, K)])

    return _sc_gather


CP = 8192
PREP_GRID = -(-FEAT_DIM // CP)
W1_ROWS = PREP_GRID * CP // 128
W1_LIN = W1_ROWS * 128

_P_np = np.zeros((128, 128), np.float32)
for _j in range(8):
    _P_np[_j * E + np.arange(E), 16 * _j + np.arange(E)] = 1.0


def _prep_body(xt_ref, w1_ref, p_ref, o_ref, w1o_ref):
    z3 = xt_ref[...].T.reshape(CP // 8, 8, E)
    acc = jnp.dot(z3[:, 0, :], p_ref[0:E, :], preferred_element_type=jnp.float32)
    for j in range(1, 8):
        acc = acc + jnp.dot(z3[:, j, :], p_ref[E * j:E * (j + 1), :],
                            preferred_element_type=jnp.float32)
    o_ref[...] = acc
    w1o_ref[...] = w1_ref[...].reshape(CP // 128, 128)


_R_np = np.kron(np.eye(F, dtype=np.float32), np.ones((1, E), dtype=np.float32))
_G_np = np.tile(np.eye(E, dtype=np.float32), (F, 1))

BM = 1024
GRID = B // BM


def _tc_body(x_ref, v_ref, w1g_ref, r_ref, g_ref, w0_ref, b0_ref, w1_ref, b1_ref,
             w2_ref, b2_ref, wo_ref, scal_ref, o_ref):
    x = x_ref[...]
    v = v_ref[...]
    w1g = w1g_ref[...]
    fm_bias = scal_ref[0, 0]
    wo0 = scal_ref[0, 1]
    bo0 = scal_ref[0, 2]
    y1 = jnp.sum(w1g * v, axis=1, keepdims=True)
    vexp = jnp.dot(v, r_ref[...], preferred_element_type=jnp.float32)
    ev = x * vexp
    s = jnp.dot(ev, g_ref[...], preferred_element_type=jnp.float32)
    sq = jnp.dot(ev * ev, g_ref[...], preferred_element_type=jnp.float32)
    y2 = 0.5 * (jnp.sum(s * s, axis=1, keepdims=True)
                - jnp.sum(sq, axis=1, keepdims=True))
    yfm = y1 + y2 + fm_bias
    h = jnp.maximum(jnp.dot(x, w0_ref[...], preferred_element_type=jnp.float32)
                    + b0_ref[...], 0.0)
    h = jnp.maximum(jnp.dot(h, w1_ref[...], preferred_element_type=jnp.float32)
                    + b1_ref[...], 0.0)
    h = jnp.maximum(jnp.dot(h, w2_ref[...], preferred_element_type=jnp.float32)
                    + b2_ref[...], 0.0)
    z = yfm * wo0 + jnp.dot(h, wo_ref[...], preferred_element_type=jnp.float32) + bo0
    o_ref[...] = jax.nn.sigmoid(z)


def kernel(feat_index, feat_value, emb_W, w1, fm_bias, W0, b0, W1, b1, W2, b2, Wo, bo):
    idx_flat = feat_index.reshape(-1).astype(jnp.int32)
    emb128, w1_128 = pl.pallas_call(
        _prep_body,
        grid=(PREP_GRID,),
        in_specs=[
            pl.BlockSpec((E, CP), lambda i: (0, i)),
            pl.BlockSpec((1, CP), lambda i: (0, i)),
            pl.BlockSpec((128, 128), lambda i: (0, 0)),
        ],
        out_specs=[
            pl.BlockSpec((CP // 8, 128), lambda i: (i, 0)),
            pl.BlockSpec((CP // 128, 128), lambda i: (i, 0)),
        ],
        out_shape=[
            jax.ShapeDtypeStruct((FEAT_DIM // 8, 128), jnp.float32),
            jax.ShapeDtypeStruct((W1_ROWS, 128), jnp.float32),
        ],
    )(emb_W.T, w1.T, jnp.asarray(_P_np))
    emb_lin = emb128.reshape(FEAT_DIM, E)
    emb_rows, w1g = _sc_gather_fn()(idx_flat, emb_lin, w1_128.reshape(W1_LIN))
    x = emb_rows.reshape(B, D_IN)
    w1g2 = w1g.reshape(B, F)
    scal = jnp.stack([fm_bias.astype(jnp.float32), Wo[0, 0], bo[0]]).reshape(1, 3)
    out = pl.pallas_call(
        _tc_body,
        grid=(GRID,),
        in_specs=[
            pl.BlockSpec((BM, D_IN), lambda i: (i, 0)),
            pl.BlockSpec((BM, F), lambda i: (i, 0)),
            pl.BlockSpec((BM, F), lambda i: (i, 0)),
            pl.BlockSpec((F, D_IN), lambda i: (0, 0)),
            pl.BlockSpec((D_IN, E), lambda i: (0, 0)),
            pl.BlockSpec((D_IN, 32), lambda i: (0, 0)),
            pl.BlockSpec((1, 32), lambda i: (0, 0)),
            pl.BlockSpec((32, 32), lambda i: (0, 0)),
            pl.BlockSpec((1, 32), lambda i: (0, 0)),
            pl.BlockSpec((32, 32), lambda i: (0, 0)),
            pl.BlockSpec((1, 32), lambda i: (0, 0)),
            pl.BlockSpec((32, 1), lambda i: (0, 0)),
            pl.BlockSpec((1, 3), lambda i: (0, 0)),
        ],
        out_specs=pl.BlockSpec((BM, 1), lambda i: (i, 0)),
        out_shape=jax.ShapeDtypeStruct((B, 1), jnp.float32),
    )(x, feat_value, w1g2, jnp.asarray(_R_np), jnp.asarray(_G_np),
      W0, b0.reshape(1, 32), W1, b1.reshape(1, 32), W2, b2.reshape(1, 32),
      Wo[1:, :], scal)
    return out
